# baseline (speedup 1.0000x reference)
import functools

import jax
import jax.numpy as jnp
from jax import lax
from jax.experimental import pallas as pl
from jax.experimental.pallas import tpu as pltpu
from jax.experimental.pallas import tpu_sc as plsc

N = 10000
E = 320000
H = 128
R = 64
L = 3
C = 10
B = 32
ZMAX = 100
CUT = 10.0

NC = 2
NS = 16
NW = NC * NS
K = 80
EW = E // NW
CH = EW // K
ZROWS = 200

_LOG2 = 0.6931471805599453


def _ssp(x):
    return jnp.maximum(x, 0.0) + jnp.log(1.0 + jnp.exp(-jnp.abs(x))) - _LOG2


def _dotb(a, b):
    return jnp.dot(a.astype(jnp.bfloat16), b.astype(jnp.bfloat16),
                   preferred_element_type=jnp.float32)


def _mesh():
    return plsc.VectorSubcoreMesh(
        core_axis_name="c", subcore_axis_name="s", num_cores=NC, num_subcores=NS
    )


_NG = EW // 16


def _posdiff_body(pos_hbm, src_hbm, dst_hbm, out_hbm, posv, idxs, idxd, d2v):
    c = lax.axis_index("c")
    s = lax.axis_index("s")
    wid = s * NC + c
    pltpu.sync_copy(pos_hbm, posv)
    pltpu.sync_copy(src_hbm.at[wid], idxs)
    pltpu.sync_copy(dst_hbm.at[wid], idxd)

    def it(i, carry):
        sl = pl.ds(i * 16, 16)
        si = idxs[sl] * 4
        di = idxd[sl] * 4
        acc = jnp.zeros((16,), jnp.float32)
        for comp in range(3):
            a = plsc.load_gather(posv, [si + comp])
            b = plsc.load_gather(posv, [di + comp])
            df = a - b
            acc = acc + df * df
        d2v[sl] = acc
        return carry

    lax.fori_loop(0, _NG, it, 0)
    pltpu.sync_copy(d2v, out_hbm.at[wid])


def _posdiff(pos4, srcw, dstw):
    return pl.kernel(
        _posdiff_body,
        out_type=jax.ShapeDtypeStruct((NW, EW), jnp.float32),
        mesh=_mesh(),
        scratch_types=[
            pltpu.VMEM((N * 4,), jnp.float32),
            pltpu.VMEM((EW,), jnp.int32),
            pltpu.VMEM((EW,), jnp.int32),
            pltpu.VMEM((EW,), jnp.float32),
        ],
        compiler_params=pltpu.CompilerParams(needs_layout_passes=False),
        name="sc_posdiff",
    )(pos4, srcw, dstw)


def _mp_body(h_hbm, w_hbm, src_hbm, dst_hbm, zero_hbm, out_hbm, idxs, idxd, hr, wr, aggS, sem):
    c = lax.axis_index("c")
    s = lax.axis_index("s")
    wid = s * NC + c
    pltpu.sync_copy(src_hbm.at[wid], idxs)
    pltpu.sync_copy(dst_hbm.at[wid], idxd)

    @pl.when(s < 10)
    def _zero():
        pltpu.sync_copy(zero_hbm.at[pl.ds(s * 1000, 1000)], aggS.at[pl.ds(s * 1000, 1000)])

    plsc.subcore_barrier()

    def chunk(j, carry):
        g = pltpu.async_copy(h_hbm.at[idxs.at[pl.ds(j * K, K)]], hr, sem)
        pltpu.sync_copy(w_hbm.at[pl.ds(wid * EW + j * K, K)], wr)
        g.wait()

        def row(k, carry2):
            for m in range(H // 16):
                sl = pl.ds(m * 16, 16)
                hr[k, sl] = hr[k, sl] * wr[k, sl]
            return carry2

        lax.fori_loop(0, K, row, 0, unroll=4)
        pltpu.sync_copy(hr, aggS.at[idxd.at[j]], add=True)
        return carry

    lax.fori_loop(0, CH, chunk, 0)
    plsc.subcore_barrier()

    @pl.when(s < 10)
    def _writeout():
        pltpu.sync_copy(aggS.at[pl.ds(s * 1000, 1000)], out_hbm.at[c, pl.ds(s * 1000, 1000)])


def _mp(h, w, srcw, dst3, zero):
    return pl.kernel(
        _mp_body,
        out_type=jax.ShapeDtypeStruct((NC, N, H), jnp.float32),
        mesh=_mesh(),
        scratch_types=[
            pltpu.VMEM((EW,), jnp.int32),
            pltpu.VMEM((CH, K), jnp.int32),
            pltpu.VMEM((K, H), jnp.float32),
            pltpu.VMEM((K, H), jnp.float32),
            pltpu.VMEM_SHARED((N, H), jnp.float32),
            pltpu.SemaphoreType.DMA,
        ],
        name="sc_mp",
        compiler_params=pltpu.CompilerParams(needs_layout_passes=False),
    )(h, w, srcw, dst3, zero)


_TE = 2000


def _filters_body(d2_ref, wf1_ref, bf1_ref, wf2_ref, bf2_ref, w0_ref, w1_ref, w2_ref):
    d2 = d2_ref[...]
    d = jnp.sqrt(d2 + 1e-12)
    offs = lax.broadcasted_iota(jnp.int32, (1, R), 1).astype(jnp.float32) * (CUT / (R - 1))
    gamma = 10.0 / CUT
    rbf = jnp.exp(-gamma * (d - offs) ** 2)
    cc = 0.5 * (jnp.cos(jnp.pi * jnp.clip(d, 0.0, CUT) / CUT) + 1.0)
    outs = (w0_ref, w1_ref, w2_ref)
    for l in range(L):
        a = _dotb(rbf, wf1_ref[l]) + bf1_ref[l]
        a = _ssp(a)
        wl = _dotb(a, wf2_ref[l]) + bf2_ref[l]
        outs[l][...] = wl * cc


def _filters(d2col, wf1s, bf1s, wf2s, bf2s):
    nblk = E // _TE
    shp = jax.ShapeDtypeStruct((E, H), jnp.float32)
    return pl.pallas_call(
        _filters_body,
        grid=(nblk,),
        in_specs=[
            pl.BlockSpec((_TE, 1), lambda i: (i, 0)),
            pl.BlockSpec((L, R, H), lambda i: (0, 0, 0)),
            pl.BlockSpec((L, 1, H), lambda i: (0, 0, 0)),
            pl.BlockSpec((L, H, H), lambda i: (0, 0, 0)),
            pl.BlockSpec((L, 1, H), lambda i: (0, 0, 0)),
        ],
        out_specs=[pl.BlockSpec((_TE, H), lambda i: (i, 0))] * L,
        out_shape=[shp, shp, shp],
        name="tc_filters",
    )(d2col, wf1s, bf1s, wf2s, bf2s)


_TN = 1000


def _x0_body(z_ref, emb_ref, wl1_ref, bl1_ref, x_ref, h_ref):
    z = z_ref[0]
    kidx = lax.broadcasted_iota(jnp.int32, (ZMAX, 1), 0)
    oht = (z == kidx).astype(jnp.float32)
    x0 = lax.dot_general(oht, emb_ref[...], (((0,), (0,)), ((), ())),
                         preferred_element_type=jnp.float32,
                         precision=lax.Precision.HIGHEST)
    x_ref[...] = x0
    h_ref[...] = _dotb(x0, wl1_ref[...]) + bl1_ref[...]


def _x0(z3, emb, wl1, bl1):
    nblk = N // _TN
    shp = jax.ShapeDtypeStruct((N, H), jnp.float32)
    return pl.pallas_call(
        _x0_body,
        grid=(nblk,),
        in_specs=[
            pl.BlockSpec((1, 1, _TN), lambda i: (i, 0, 0)),
            pl.BlockSpec((ZMAX, H), lambda i: (0, 0)),
            pl.BlockSpec((H, H), lambda i: (0, 0)),
            pl.BlockSpec((1, H), lambda i: (0, 0)),
        ],
        out_specs=[pl.BlockSpec((_TN, H), lambda i: (i, 0))] * 2,
        out_shape=[shp, shp],
        name="tc_x0",
    )(z3, emb, wl1, bl1)


def _update_body(x_ref, a0_ref, a1_ref, wl2_ref, bl2_ref, wl1_ref, bl1_ref, xn_ref, hn_ref):
    sagg = _ssp(a0_ref[0] + a1_ref[0])
    xn = x_ref[...] + _dotb(sagg, wl2_ref[...]) + bl2_ref[...]
    xn_ref[...] = xn
    hn_ref[...] = _dotb(xn, wl1_ref[...]) + bl1_ref[...]


def _update(x, agg2, wl2, bl2, wl1n, bl1n):
    nblk = N // _TN
    shp = jax.ShapeDtypeStruct((N, H), jnp.float32)
    return pl.pallas_call(
        _update_body,
        grid=(nblk,),
        in_specs=[
            pl.BlockSpec((_TN, H), lambda i: (i, 0)),
            pl.BlockSpec((1, _TN, H), lambda i: (0, i, 0)),
            pl.BlockSpec((1, _TN, H), lambda i: (1, i, 0)),
            pl.BlockSpec((H, H), lambda i: (0, 0)),
            pl.BlockSpec((1, H), lambda i: (0, 0)),
            pl.BlockSpec((H, H), lambda i: (0, 0)),
            pl.BlockSpec((1, H), lambda i: (0, 0)),
        ],
        out_specs=[pl.BlockSpec((_TN, H), lambda i: (i, 0))] * 2,
        out_shape=[shp, shp],
        name="tc_update",
    )(x, agg2, agg2, wl2, bl2, wl1n, bl1n)


def _readout_body(x_ref, a0_ref, a1_ref, wl2_ref, bl2_ref, wo1_ref, bo1_ref,
                  wo2_ref, bo2_ref, batch_ref, g_ref):
    i = pl.program_id(0)

    @pl.when(i == 0)
    def _init():
        g_ref[...] = jnp.zeros_like(g_ref)

    sagg = _ssp(a0_ref[0] + a1_ref[0])
    xf = x_ref[...] + _dotb(sagg, wl2_ref[...]) + bl2_ref[...]
    hm = _ssp(_dotb(xf, wo1_ref[...]) + bo1_ref[...])
    no = _dotb(hm, wo2_ref[...]) + bo2_ref[...]
    b = batch_ref[0]
    kidx = lax.broadcasted_iota(jnp.int32, (B, 1), 0)
    oht = (b == kidx).astype(jnp.float32)
    g_ref[...] += jnp.dot(oht, no, preferred_element_type=jnp.float32,
                          precision=lax.Precision.HIGHEST)


def _readout(x, agg2, wl2, bl2, wo1, bo1, wo2p, bo2p, batch3):
    nblk = N // _TN
    return pl.pallas_call(
        _readout_body,
        grid=(nblk,),
        in_specs=[
            pl.BlockSpec((_TN, H), lambda i: (i, 0)),
            pl.BlockSpec((1, _TN, H), lambda i: (0, i, 0)),
            pl.BlockSpec((1, _TN, H), lambda i: (1, i, 0)),
            pl.BlockSpec((H, H), lambda i: (0, 0)),
            pl.BlockSpec((1, H), lambda i: (0, 0)),
            pl.BlockSpec((H, H // 2), lambda i: (0, 0)),
            pl.BlockSpec((1, H // 2), lambda i: (0, 0)),
            pl.BlockSpec((H // 2, H), lambda i: (0, 0)),
            pl.BlockSpec((1, H), lambda i: (0, 0)),
            pl.BlockSpec((1, 1, _TN), lambda i: (i, 0, 0)),
        ],
        out_specs=pl.BlockSpec((B, H), lambda i: (0, 0)),
        out_shape=jax.ShapeDtypeStruct((B, H), jnp.float32),
        name="tc_readout",
    )(x, agg2, agg2, wl2, bl2, wo1, bo1, wo2p, bo2p, batch3)


def _heads_body(g_ref, wc_ref, bc_ref, wr_ref, br_ref, cp_ref, rp_ref):
    g = g_ref[...]
    logits = _dotb(g, wc_ref[...]) + bc_ref[...]
    lane = lax.broadcasted_iota(jnp.int32, (1, H), 1)
    lm = jnp.where(lane < C, logits, -1e30)
    m = jnp.max(lm, axis=1, keepdims=True)
    p = jnp.exp(lm - m)
    p = jnp.where(lane < C, p, 0.0)
    cp_ref[...] = p / jnp.sum(p, axis=1, keepdims=True)
    rp_ref[...] = _dotb(g, wr_ref[...]) + br_ref[...]


def _heads(g, wcp, bcp, wrp, brp):
    shp = jax.ShapeDtypeStruct((B, H), jnp.float32)
    return pl.pallas_call(
        _heads_body,
        grid=(1,),
        in_specs=[pl.BlockSpec((B, H), lambda i: (0, 0)),
                  pl.BlockSpec((H, H), lambda i: (0, 0)),
                  pl.BlockSpec((1, H), lambda i: (0, 0)),
                  pl.BlockSpec((H, H), lambda i: (0, 0)),
                  pl.BlockSpec((1, H), lambda i: (0, 0))],
        out_specs=[pl.BlockSpec((B, H), lambda i: (0, 0))] * 2,
        out_shape=[shp, shp],
        name="tc_heads",
    )(g, wcp, bcp, wrp, brp)


def kernel(z, pos, edge_index, batch, emb, params):
    src = edge_index[0].astype(jnp.int32)
    dst = edge_index[1].astype(jnp.int32)
    srcw = src.reshape(NW, EW)
    dstw = dst.reshape(NW, EW)
    dst3 = dst.reshape(NW, CH, K)
    zero = jnp.zeros((N, H), jnp.float32)
    pos4 = jnp.zeros((N, 4), jnp.float32).at[:, :3].set(pos).reshape(-1)
    z3 = z.astype(jnp.int32).reshape(N // _TN, 1, _TN)
    batch3 = batch.astype(jnp.int32).reshape(N // _TN, 1, _TN)

    wf1s = jnp.stack([params[f"Wf1_{l}"] for l in range(L)])
    bf1s = jnp.stack([params[f"bf1_{l}"].reshape(1, H) for l in range(L)])
    wf2s = jnp.stack([params[f"Wf2_{l}"] for l in range(L)])
    bf2s = jnp.stack([params[f"bf2_{l}"].reshape(1, H) for l in range(L)])

    wo2p = jnp.zeros((H // 2, H), jnp.float32).at[:, : C + 1].set(params["Wo2"])
    bo2p = jnp.zeros((1, H), jnp.float32).at[0, : C + 1].set(params["bo2"])
    wcp = jnp.zeros((H, H), jnp.float32).at[: C + 1, :C].set(params["Wc"])
    bcp = jnp.zeros((1, H), jnp.float32).at[0, :C].set(params["bc"])
    wrp = jnp.zeros((H, H), jnp.float32).at[: C + 1, :1].set(params["Wr"])
    brp = jnp.zeros((1, H), jnp.float32).at[0, :1].set(params["br"])

    d2col = _posdiff(pos4, srcw, dstw).reshape(E, 1)
    w0, w1, w2 = _filters(d2col, wf1s, bf1s, wf2s, bf2s)
    ws = (w0, w1, w2)

    x, h = _x0(z3, emb, params["Wl1_0"], params["bl1_0"].reshape(1, H))
    for l in range(L):
        agg2 = _mp(h, ws[l], srcw, dst3, zero)
        if l < L - 1:
            x, h = _update(
                x, agg2,
                params[f"Wl2_{l}"], params[f"bl2_{l}"].reshape(1, H),
                params[f"Wl1_{l + 1}"], params[f"bl1_{l + 1}"].reshape(1, H),
            )
        else:
            g = _readout(
                x, agg2,
                params[f"Wl2_{l}"], params[f"bl2_{l}"].reshape(1, H),
                params["Wo1"], params["bo1"].reshape(1, H // 2),
                wo2p, bo2p, batch3,
            )
    cp, rp = _heads(g, wcp, bcp, wrp, brp)
    return cp[:, :C], rp[:, :1]


# speedup vs baseline: 1.7359x; 1.7359x over previous
import functools

import jax
import jax.numpy as jnp
from jax import lax
from jax.experimental import pallas as pl
from jax.experimental.pallas import tpu as pltpu
from jax.experimental.pallas import tpu_sc as plsc

N = 10000
E = 320000
H = 128
R = 64
L = 3
C = 10
B = 32
ZMAX = 100
CUT = 10.0

NC = 2
NS = 16
NW = NC * NS
K = 80
EW = E // NW
CH = EW // K
ZROWS = 200

_LOG2 = 0.6931471805599453


def _ssp(x):
    return jnp.maximum(x, 0.0) + jnp.log(1.0 + jnp.exp(-jnp.abs(x))) - _LOG2


def _dotb(a, b):
    return jnp.dot(a.astype(jnp.bfloat16), b.astype(jnp.bfloat16),
                   preferred_element_type=jnp.float32)


def _mesh():
    return plsc.VectorSubcoreMesh(
        core_axis_name="c", subcore_axis_name="s", num_cores=NC, num_subcores=NS
    )


_NG = EW // 16


def _posdiff_body(pos_hbm, src_hbm, dst_hbm, out_hbm, posv, idxs, idxd, d2v):
    c = lax.axis_index("c")
    s = lax.axis_index("s")
    wid = s * NC + c
    pltpu.sync_copy(pos_hbm, posv)
    pltpu.sync_copy(src_hbm.at[wid], idxs)
    pltpu.sync_copy(dst_hbm.at[wid], idxd)

    def it(i, carry):
        sl = pl.ds(i * 16, 16)
        si = idxs[sl] * 4
        di = idxd[sl] * 4
        acc = jnp.zeros((16,), jnp.float32)
        for comp in range(3):
            a = plsc.load_gather(posv, [si + comp])
            b = plsc.load_gather(posv, [di + comp])
            df = a - b
            acc = acc + df * df
        d2v[sl] = acc
        return carry

    lax.fori_loop(0, _NG, it, 0)
    pltpu.sync_copy(d2v, out_hbm.at[wid])


def _posdiff(pos4, srcw, dstw):
    return pl.kernel(
        _posdiff_body,
        out_type=jax.ShapeDtypeStruct((NW, EW), jnp.float32),
        mesh=_mesh(),
        scratch_types=[
            pltpu.VMEM((N * 4,), jnp.float32),
            pltpu.VMEM((EW,), jnp.int32),
            pltpu.VMEM((EW,), jnp.int32),
            pltpu.VMEM((EW,), jnp.float32),
        ],
        compiler_params=pltpu.CompilerParams(needs_layout_passes=False),
        name="sc_posdiff",
    )(pos4, srcw, dstw)


def _mp_body(h_hbm, w_hbm, src_hbm, dst_hbm, zero_hbm, out_hbm, idxs, idxd, hr, wr, aggS, sem):
    c = lax.axis_index("c")
    s = lax.axis_index("s")
    wid = s * NC + c
    pltpu.sync_copy(src_hbm.at[wid], idxs)
    pltpu.sync_copy(dst_hbm.at[wid], idxd)

    @pl.when(s < 10)
    def _zero():
        pltpu.sync_copy(zero_hbm.at[pl.ds(s * 1000, 1000)], aggS.at[pl.ds(s * 1000, 1000)])

    plsc.subcore_barrier()

    def chunk(j, carry):
        g = pltpu.async_copy(h_hbm.at[idxs.at[pl.ds(j * K, K)]], hr, sem)
        pltpu.sync_copy(w_hbm.at[pl.ds(wid * EW + j * K, K)], wr)
        g.wait()

        def row(k, carry2):
            def sub(m, carry3):
                sl = pl.ds(m * 16, 16)
                hr[k, sl] = hr[k, sl] * wr[k, sl]
                return carry3

            lax.fori_loop(0, H // 16, sub, 0)
            return carry2

        lax.fori_loop(0, K, row, 0)
        pltpu.sync_copy(hr, aggS.at[idxd.at[j]], add=True)
        return carry

    lax.fori_loop(0, CH, chunk, 0)
    plsc.subcore_barrier()

    @pl.when(s < 10)
    def _writeout():
        pltpu.sync_copy(aggS.at[pl.ds(s * 1000, 1000)], out_hbm.at[c, pl.ds(s * 1000, 1000)])


def _mp(h, w, srcw, dst3, zero):
    return pl.kernel(
        _mp_body,
        out_type=jax.ShapeDtypeStruct((NC, N, H), jnp.float32),
        mesh=_mesh(),
        scratch_types=[
            pltpu.VMEM((EW,), jnp.int32),
            pltpu.VMEM((CH, K), jnp.int32),
            pltpu.VMEM((K, H), jnp.float32),
            pltpu.VMEM((K, H), jnp.float32),
            pltpu.VMEM_SHARED((N, H), jnp.float32),
            pltpu.SemaphoreType.DMA,
        ],
        name="sc_mp",
        compiler_params=pltpu.CompilerParams(needs_layout_passes=False),
    )(h, w, srcw, dst3, zero)


_TE = 2000


def _filters_body(nl, d2_ref, wf1_ref, bf1_ref, wf2_ref, bf2_ref, *out_refs):
    d2row = d2_ref[0]
    drow = jnp.sqrt(d2row + 1e-12)
    ccrow = 0.5 * (jnp.cos(jnp.pi * jnp.clip(drow, 0.0, CUT) / CUT) + 1.0)
    ones_r = jnp.ones((1, R), jnp.float32)
    ones_h = jnp.ones((1, H), jnp.float32)
    dn = (((0,), (0,)), ((), ()))
    dmat = lax.dot_general(drow, ones_r, dn, precision=lax.Precision.HIGHEST,
                           preferred_element_type=jnp.float32)
    ccmat = lax.dot_general(ccrow, ones_h, dn, precision=lax.Precision.HIGHEST,
                            preferred_element_type=jnp.float32)
    offs = lax.broadcasted_iota(jnp.int32, (1, R), 1).astype(jnp.float32) * (CUT / (R - 1))
    gamma = 10.0 / CUT
    rbf = jnp.exp(-gamma * (dmat - offs) ** 2)
    for l in range(nl):
        a = _dotb(rbf, wf1_ref[l]) + bf1_ref[l]
        a = _ssp(a)
        wl = _dotb(a, wf2_ref[l]) + bf2_ref[l]
        out_refs[l][...] = wl * ccmat


def _filters(d23, wf1s, bf1s, wf2s, bf2s, name):
    nl = wf1s.shape[0]
    nblk = E // _TE
    shp = jax.ShapeDtypeStruct((E, H), jnp.float32)
    return pl.pallas_call(
        functools.partial(_filters_body, nl),
        grid=(nblk,),
        in_specs=[
            pl.BlockSpec((1, 1, _TE), lambda i: (i, 0, 0)),
            pl.BlockSpec((nl, R, H), lambda i: (0, 0, 0)),
            pl.BlockSpec((nl, 1, H), lambda i: (0, 0, 0)),
            pl.BlockSpec((nl, H, H), lambda i: (0, 0, 0)),
            pl.BlockSpec((nl, 1, H), lambda i: (0, 0, 0)),
        ],
        out_specs=[pl.BlockSpec((_TE, H), lambda i: (i, 0))] * nl,
        out_shape=[shp] * nl,
        name=name,
    )(d23, wf1s, bf1s, wf2s, bf2s)


_TN = 1000


def _x0_body(z_ref, emb_ref, wl1_ref, bl1_ref, x_ref, h_ref):
    z = z_ref[0]
    kidx = lax.broadcasted_iota(jnp.int32, (ZMAX, 1), 0)
    oht = (z == kidx).astype(jnp.float32)
    x0 = lax.dot_general(oht, emb_ref[...], (((0,), (0,)), ((), ())),
                         preferred_element_type=jnp.float32,
                         precision=lax.Precision.HIGHEST)
    x_ref[...] = x0
    h_ref[...] = _dotb(x0, wl1_ref[...]) + bl1_ref[...]


def _x0(z3, emb, wl1, bl1):
    nblk = N // _TN
    shp = jax.ShapeDtypeStruct((N, H), jnp.float32)
    return pl.pallas_call(
        _x0_body,
        grid=(nblk,),
        in_specs=[
            pl.BlockSpec((1, 1, _TN), lambda i: (i, 0, 0)),
            pl.BlockSpec((ZMAX, H), lambda i: (0, 0)),
            pl.BlockSpec((H, H), lambda i: (0, 0)),
            pl.BlockSpec((1, H), lambda i: (0, 0)),
        ],
        out_specs=[pl.BlockSpec((_TN, H), lambda i: (i, 0))] * 2,
        out_shape=[shp, shp],
        name="tc_x0",
    )(z3, emb, wl1, bl1)


def _update_body(x_ref, a0_ref, a1_ref, wl2_ref, bl2_ref, wl1_ref, bl1_ref, xn_ref, hn_ref):
    sagg = _ssp(a0_ref[0] + a1_ref[0])
    xn = x_ref[...] + _dotb(sagg, wl2_ref[...]) + bl2_ref[...]
    xn_ref[...] = xn
    hn_ref[...] = _dotb(xn, wl1_ref[...]) + bl1_ref[...]


def _update(x, agg2, wl2, bl2, wl1n, bl1n):
    nblk = N // _TN
    shp = jax.ShapeDtypeStruct((N, H), jnp.float32)
    return pl.pallas_call(
        _update_body,
        grid=(nblk,),
        in_specs=[
            pl.BlockSpec((_TN, H), lambda i: (i, 0)),
            pl.BlockSpec((1, _TN, H), lambda i: (0, i, 0)),
            pl.BlockSpec((1, _TN, H), lambda i: (1, i, 0)),
            pl.BlockSpec((H, H), lambda i: (0, 0)),
            pl.BlockSpec((1, H), lambda i: (0, 0)),
            pl.BlockSpec((H, H), lambda i: (0, 0)),
            pl.BlockSpec((1, H), lambda i: (0, 0)),
        ],
        out_specs=[pl.BlockSpec((_TN, H), lambda i: (i, 0))] * 2,
        out_shape=[shp, shp],
        name="tc_update",
    )(x, agg2, agg2, wl2, bl2, wl1n, bl1n)


def _readout_body(x_ref, a0_ref, a1_ref, wl2_ref, bl2_ref, wo1_ref, bo1_ref,
                  wo2_ref, bo2_ref, batch_ref, g_ref):
    i = pl.program_id(0)

    @pl.when(i == 0)
    def _init():
        g_ref[...] = jnp.zeros_like(g_ref)

    sagg = _ssp(a0_ref[0] + a1_ref[0])
    xf = x_ref[...] + _dotb(sagg, wl2_ref[...]) + bl2_ref[...]
    hm = _ssp(_dotb(xf, wo1_ref[...]) + bo1_ref[...])
    no = _dotb(hm, wo2_ref[...]) + bo2_ref[...]
    b = batch_ref[0]
    kidx = lax.broadcasted_iota(jnp.int32, (B, 1), 0)
    oht = (b == kidx).astype(jnp.float32)
    g_ref[...] += jnp.dot(oht, no, preferred_element_type=jnp.float32,
                          precision=lax.Precision.HIGHEST)


def _readout(x, agg2, wl2, bl2, wo1, bo1, wo2p, bo2p, batch3):
    nblk = N // _TN
    return pl.pallas_call(
        _readout_body,
        grid=(nblk,),
        in_specs=[
            pl.BlockSpec((_TN, H), lambda i: (i, 0)),
            pl.BlockSpec((1, _TN, H), lambda i: (0, i, 0)),
            pl.BlockSpec((1, _TN, H), lambda i: (1, i, 0)),
            pl.BlockSpec((H, H), lambda i: (0, 0)),
            pl.BlockSpec((1, H), lambda i: (0, 0)),
            pl.BlockSpec((H, H // 2), lambda i: (0, 0)),
            pl.BlockSpec((1, H // 2), lambda i: (0, 0)),
            pl.BlockSpec((H // 2, H), lambda i: (0, 0)),
            pl.BlockSpec((1, H), lambda i: (0, 0)),
            pl.BlockSpec((1, 1, _TN), lambda i: (i, 0, 0)),
        ],
        out_specs=pl.BlockSpec((B, H), lambda i: (0, 0)),
        out_shape=jax.ShapeDtypeStruct((B, H), jnp.float32),
        name="tc_readout",
    )(x, agg2, agg2, wl2, bl2, wo1, bo1, wo2p, bo2p, batch3)


def _heads_body(g_ref, wc_ref, bc_ref, wr_ref, br_ref, cp_ref, rp_ref):
    g = g_ref[...]
    logits = _dotb(g, wc_ref[...]) + bc_ref[...]
    lane = lax.broadcasted_iota(jnp.int32, (1, H), 1)
    lm = jnp.where(lane < C, logits, -1e30)
    m = jnp.max(lm, axis=1, keepdims=True)
    p = jnp.exp(lm - m)
    p = jnp.where(lane < C, p, 0.0)
    cp_ref[...] = p / jnp.sum(p, axis=1, keepdims=True)
    rp_ref[...] = _dotb(g, wr_ref[...]) + br_ref[...]


def _heads(g, wcp, bcp, wrp, brp):
    shp = jax.ShapeDtypeStruct((B, H), jnp.float32)
    return pl.pallas_call(
        _heads_body,
        grid=(1,),
        in_specs=[pl.BlockSpec((B, H), lambda i: (0, 0)),
                  pl.BlockSpec((H, H), lambda i: (0, 0)),
                  pl.BlockSpec((1, H), lambda i: (0, 0)),
                  pl.BlockSpec((H, H), lambda i: (0, 0)),
                  pl.BlockSpec((1, H), lambda i: (0, 0))],
        out_specs=[pl.BlockSpec((B, H), lambda i: (0, 0))] * 2,
        out_shape=[shp, shp],
        name="tc_heads",
    )(g, wcp, bcp, wrp, brp)


def kernel(z, pos, edge_index, batch, emb, params):
    src = edge_index[0].astype(jnp.int32)
    dst = edge_index[1].astype(jnp.int32)
    srcw = src.reshape(NW, EW)
    dstw = dst.reshape(NW, EW)
    dst3 = dst.reshape(NW, CH, K)
    zero = jnp.zeros((N, H), jnp.float32)
    pos4 = jnp.zeros((N, 4), jnp.float32).at[:, :3].set(pos).reshape(-1)
    z3 = z.astype(jnp.int32).reshape(N // _TN, 1, _TN)
    batch3 = batch.astype(jnp.int32).reshape(N // _TN, 1, _TN)

    wf1s = jnp.stack([params[f"Wf1_{l}"] for l in range(L)])
    bf1s = jnp.stack([params[f"bf1_{l}"].reshape(1, H) for l in range(L)])
    wf2s = jnp.stack([params[f"Wf2_{l}"] for l in range(L)])
    bf2s = jnp.stack([params[f"bf2_{l}"].reshape(1, H) for l in range(L)])

    wo2p = jnp.zeros((H // 2, H), jnp.float32).at[:, : C + 1].set(params["Wo2"])
    bo2p = jnp.zeros((1, H), jnp.float32).at[0, : C + 1].set(params["bo2"])
    wcp = jnp.zeros((H, H), jnp.float32).at[: C + 1, :C].set(params["Wc"])
    bcp = jnp.zeros((1, H), jnp.float32).at[0, :C].set(params["bc"])
    wrp = jnp.zeros((H, H), jnp.float32).at[: C + 1, :1].set(params["Wr"])
    brp = jnp.zeros((1, H), jnp.float32).at[0, :1].set(params["br"])

    d23 = _posdiff(pos4, srcw, dstw).reshape(E // _TE, 1, _TE)
    (w0,) = _filters(d23, wf1s[:1], bf1s[:1], wf2s[:1], bf2s[:1], "tc_filters0")
    x, h = _x0(z3, emb, params["Wl1_0"], params["bl1_0"].reshape(1, H))
    w12 = _filters(d23, wf1s[1:], bf1s[1:], wf2s[1:], bf2s[1:], "tc_filters12")
    ws = (w0,) + tuple(w12)
    for l in range(L):
        agg2 = _mp(h, ws[l], srcw, dst3, zero)
        if l < L - 1:
            x, h = _update(
                x, agg2,
                params[f"Wl2_{l}"], params[f"bl2_{l}"].reshape(1, H),
                params[f"Wl1_{l + 1}"], params[f"bl1_{l + 1}"].reshape(1, H),
            )
        else:
            g = _readout(
                x, agg2,
                params[f"Wl2_{l}"], params[f"bl2_{l}"].reshape(1, H),
                params["Wo1"], params["bo1"].reshape(1, H // 2),
                wo2p, bo2p, batch3,
            )
    cp, rp = _heads(g, wcp, bcp, wrp, brp)
    return cp[:, :C], rp[:, :1]
